# baseline (speedup 1.0000x reference)
import dataclasses
import functools

import jax
import jax.numpy as jnp
from jax import lax
from jax.experimental import pallas as pl
from jax.experimental.pallas import tpu as pltpu
from jax.experimental.pallas import tpu_sc as plsc

B = 1024
IN_FEATURES = 4096
OUT_FEATURES = 2048
K = 32
NC = 2
NS = 16
NW = NC * NS
RPW = OUT_FEATURES // NW
L = 16
IW = IN_FEATURES // 2

TO = 128


def _compiler_params():
    cp = pltpu.CompilerParams()
    if "needs_layout_passes" in pltpu.CompilerParams.__dataclass_fields__:
        cp = dataclasses.replace(cp, needs_layout_passes=False)
    return cp


def _sc_scatter_weights_packed(idx, w):
    mesh = plsc.VectorSubcoreMesh(core_axis_name="c", subcore_axis_name="s")

    @functools.partial(
        pl.kernel,
        out_type=jax.ShapeDtypeStruct((OUT_FEATURES, IW), jnp.int32),
        mesh=mesh,
        compiler_params=_compiler_params(),
        scratch_types=[
            pltpu.VMEM((RPW, K), jnp.int32),
            pltpu.VMEM((RPW, K), jnp.float32),
            pltpu.VMEM((IN_FEATURES,), jnp.float32),
            pltpu.VMEM((IW,), jnp.int32),
            pltpu.VMEM((IW,), jnp.int32),
            pltpu.SemaphoreType.DMA,
            pltpu.SemaphoreType.DMA,
        ],
    )
    def body(idx_hbm, w_hbm, s_hbm, idx_v, w_v, fb_v, pb0_v, pb1_v,
             sem0, sem1):
        pbufs = (pb0_v, pb1_v)
        sems = (sem0, sem1)
        wid = lax.axis_index("s") * NC + lax.axis_index("c")
        base = wid * RPW
        pltpu.sync_copy(idx_hbm.at[pl.ds(base, RPW)], idx_v)
        pltpu.sync_copy(w_hbm.at[pl.ds(base, RPW)], w_v)

        zf = jnp.zeros((L,), jnp.float32)
        zi = jnp.zeros((L,), jnp.int32)

        @pl.loop(0, IN_FEATURES, step=2 * L)
        def _(c):
            fb_v[pl.ds(c, L)] = zf
            fb_v[pl.ds(c + L, L)] = zf
            pb0_v[pl.ds(lax.div(c, 2), L)] = zi
            pb1_v[pl.ds(lax.div(c, 2), L)] = zi

        def halves(r):
            return [idx_v[r, pl.ds(h * L, L)] for h in range(K // L)]

        def build_row(r, p):
            pb = pbufs[p]
            ivs = halves(r)
            wvs = [w_v[r, pl.ds(h * L, L)] for h in range(K // L)]
            for iv, wv in zip(ivs, wvs):
                plsc.addupdate_scatter(fb_v, [iv], wv)
            for iv in ivs:
                vals = plsc.load_gather(fb_v, [iv])
                plsc.store_scatter(fb_v, [iv], zf)
                u = plsc.bitcast(vals, jnp.int32)
                lsb = (u >> 16) & 1
                bits = ((u + 0x7FFF + lsb) >> 16) & 0xFFFF
                word = iv & (IW - 1)
                hi = iv >> 11
                shifted = jnp.where(hi == 1, bits << 16, bits)
                for parity in range(2):
                    m = hi == parity
                    cur = plsc.load_gather(pb, [word])
                    plsc.store_scatter(pb, [word], cur | shifted, mask=m)

        def unbuild_row(r, p):
            for iv in halves(r):
                plsc.store_scatter(pbufs[p], [iv & (IW - 1)], zi)

        def start_dma(r, p):
            pltpu.async_copy(pbufs[p], s_hbm.at[base + r], sems[p])

        def wait_dma(r, p):
            pltpu.make_async_copy(pbufs[p], s_hbm.at[base + r],
                                  sems[p]).wait()

        build_row(0, 0)
        start_dma(0, 0)
        build_row(1, 1)
        start_dma(1, 1)

        @pl.loop(2, RPW, step=2)
        def _(r0):
            for p in range(2):
                r = r0 + p
                wait_dma(r - 2, p)
                unbuild_row(r - 2, p)
                build_row(r, p)
                start_dma(r, p)

        wait_dma(RPW - 2, 0)
        wait_dma(RPW - 1, 1)

    return body(idx, w)


def _tc_matmul(x, s_packed, bias_row):
    NT = (((1,), (1,)), ((), ()))

    def body(x_ref, s_ref, b_ref, o_ref, xbf_ref):
        @pl.when(pl.program_id(0) == 0)
        def _():
            xbf_ref[...] = x_ref[...].astype(jnp.bfloat16)

        u = s_ref[...]
        lo = lax.bitcast_convert_type(u << 16, jnp.float32)
        hi = lax.bitcast_convert_type(u & jnp.int32(-65536), jnp.float32)
        acc = lax.dot_general(xbf_ref[:, :IW], lo.astype(jnp.bfloat16), NT,
                              preferred_element_type=jnp.float32)
        acc += lax.dot_general(xbf_ref[:, IW:], hi.astype(jnp.bfloat16), NT,
                               preferred_element_type=jnp.float32)
        o_ref[...] = acc + b_ref[...]

    return pl.pallas_call(
        body,
        grid=(OUT_FEATURES // TO,),
        in_specs=[
            pl.BlockSpec((B, IN_FEATURES), lambda j: (0, 0)),
            pl.BlockSpec((TO, IW), lambda j: (j, 0)),
            pl.BlockSpec((1, TO), lambda j: (0, j)),
        ],
        out_specs=pl.BlockSpec((B, TO), lambda j: (0, j)),
        out_shape=jax.ShapeDtypeStruct((B, OUT_FEATURES), jnp.float32),
        scratch_shapes=[pltpu.VMEM((B, IN_FEATURES), jnp.bfloat16)],
    )(x, s_packed, bias_row)


def kernel(x, indices, weight, bias):
    idx = indices.astype(jnp.int32)
    s_packed = _sc_scatter_weights_packed(idx, weight.astype(jnp.float32))
    return _tc_matmul(x, s_packed,
                      bias.astype(jnp.float32).reshape(1, OUT_FEATURES))


# speedup vs baseline: 1.2785x; 1.2785x over previous
import dataclasses
import functools

import jax
import jax.numpy as jnp
from jax import lax
from jax.experimental import pallas as pl
from jax.experimental.pallas import tpu as pltpu
from jax.experimental.pallas import tpu_sc as plsc

B = 1024
IN_FEATURES = 4096
OUT_FEATURES = 2048
K = 32
NC = 2
NS = 16
NW = NC * NS
RPW = OUT_FEATURES // NW
L = 16
IW = IN_FEATURES // 2

TO = 256


def _compiler_params():
    cp = pltpu.CompilerParams()
    if "needs_layout_passes" in pltpu.CompilerParams.__dataclass_fields__:
        cp = dataclasses.replace(cp, needs_layout_passes=False)
    return cp


def _sc_scatter_weights_packed(idx, w):
    mesh = plsc.VectorSubcoreMesh(core_axis_name="c", subcore_axis_name="s")

    @functools.partial(
        pl.kernel,
        out_type=jax.ShapeDtypeStruct((OUT_FEATURES, IW), jnp.int32),
        mesh=mesh,
        compiler_params=_compiler_params(),
        scratch_types=[
            pltpu.VMEM((RPW, K), jnp.int32),
            pltpu.VMEM((RPW, K), jnp.float32),
            pltpu.VMEM((IN_FEATURES,), jnp.float32),
            pltpu.VMEM((IW,), jnp.int32),
            pltpu.VMEM((IW,), jnp.int32),
            pltpu.SemaphoreType.DMA,
            pltpu.SemaphoreType.DMA,
        ],
    )
    def body(idx_hbm, w_hbm, s_hbm, idx_v, w_v, fb_v, pb0_v, pb1_v,
             sem0, sem1):
        pbufs = (pb0_v, pb1_v)
        sems = (sem0, sem1)
        wid = lax.axis_index("s") * NC + lax.axis_index("c")
        base = wid * RPW
        pltpu.sync_copy(idx_hbm.at[pl.ds(base, RPW)], idx_v)
        pltpu.sync_copy(w_hbm.at[pl.ds(base, RPW)], w_v)

        zf = jnp.zeros((L,), jnp.float32)
        zi = jnp.zeros((L,), jnp.int32)

        @pl.loop(0, IN_FEATURES, step=2 * L)
        def _(c):
            fb_v[pl.ds(c, L)] = zf
            fb_v[pl.ds(c + L, L)] = zf
            pb0_v[pl.ds(lax.div(c, 2), L)] = zi
            pb1_v[pl.ds(lax.div(c, 2), L)] = zi

        def halves(r):
            return [idx_v[r, pl.ds(h * L, L)] for h in range(K // L)]

        def build_row(r, p):
            pb = pbufs[p]
            ivs = halves(r)
            wvs = [w_v[r, pl.ds(h * L, L)] for h in range(K // L)]
            for iv, wv in zip(ivs, wvs):
                plsc.addupdate_scatter(fb_v, [iv], wv)
            for iv in ivs:
                vals = plsc.load_gather(fb_v, [iv])
                plsc.store_scatter(fb_v, [iv], zf)
                u = plsc.bitcast(vals, jnp.int32)
                lsb = (u >> 16) & 1
                bits = ((u + 0x7FFF + lsb) >> 16) & 0xFFFF
                word = iv & (IW - 1)
                hi = iv >> 11
                shifted = jnp.where(hi == 1, bits << 16, bits)
                for parity in range(2):
                    m = hi == parity
                    cur = plsc.load_gather(pb, [word])
                    plsc.store_scatter(pb, [word], cur | shifted, mask=m)

        def unbuild_row(r, p):
            for iv in halves(r):
                plsc.store_scatter(pbufs[p], [iv & (IW - 1)], zi)

        def start_dma(r, p):
            pltpu.async_copy(pbufs[p], s_hbm.at[base + r], sems[p])

        def wait_dma(r, p):
            pltpu.make_async_copy(pbufs[p], s_hbm.at[base + r],
                                  sems[p]).wait()

        build_row(0, 0)
        start_dma(0, 0)
        build_row(1, 1)
        start_dma(1, 1)

        @pl.loop(2, RPW, step=2)
        def _(r0):
            for p in range(2):
                r = r0 + p
                wait_dma(r - 2, p)
                unbuild_row(r - 2, p)
                build_row(r, p)
                start_dma(r, p)

        wait_dma(RPW - 2, 0)
        wait_dma(RPW - 1, 1)

    return body(idx, w)


def _tc_convert_x(x):
    def body(x_ref, o_ref):
        o_ref[...] = x_ref[...].astype(jnp.bfloat16)

    return pl.pallas_call(
        body,
        grid=(8,),
        in_specs=[pl.BlockSpec((B // 8, IN_FEATURES), lambda j: (j, 0))],
        out_specs=pl.BlockSpec((B // 8, IN_FEATURES), lambda j: (j, 0)),
        out_shape=jax.ShapeDtypeStruct((B, IN_FEATURES), jnp.bfloat16),
    )(x)


def _tc_matmul(x_bf, s_packed, bias_row):
    NT = (((1,), (1,)), ((), ()))

    def body(x_ref, s_ref, b_ref, o_ref):
        u = s_ref[...]
        lo = lax.bitcast_convert_type(u << 16, jnp.float32)
        hi = lax.bitcast_convert_type(u & jnp.int32(-65536), jnp.float32)
        acc = lax.dot_general(x_ref[:, :IW], lo.astype(jnp.bfloat16), NT,
                              preferred_element_type=jnp.float32)
        acc += lax.dot_general(x_ref[:, IW:], hi.astype(jnp.bfloat16), NT,
                               preferred_element_type=jnp.float32)
        o_ref[...] = acc + b_ref[...]

    return pl.pallas_call(
        body,
        grid=(OUT_FEATURES // TO,),
        in_specs=[
            pl.BlockSpec((B, IN_FEATURES), lambda j: (0, 0)),
            pl.BlockSpec((TO, IW), lambda j: (j, 0)),
            pl.BlockSpec((1, TO), lambda j: (0, j)),
        ],
        out_specs=pl.BlockSpec((B, TO), lambda j: (0, j)),
        out_shape=jax.ShapeDtypeStruct((B, OUT_FEATURES), jnp.float32),
    )(x_bf, s_packed, bias_row)


def kernel(x, indices, weight, bias):
    idx = indices.astype(jnp.int32)
    s_packed = _sc_scatter_weights_packed(idx, weight.astype(jnp.float32))
    x_bf = _tc_convert_x(x)
    return _tc_matmul(x_bf, s_packed,
                      bias.astype(jnp.float32).reshape(1, OUT_FEATURES))


# speedup vs baseline: 1.3174x; 1.0304x over previous
import dataclasses
import functools

import jax
import jax.numpy as jnp
from jax import lax
from jax.experimental import pallas as pl
from jax.experimental.pallas import tpu as pltpu
from jax.experimental.pallas import tpu_sc as plsc

B = 1024
IN_FEATURES = 4096
OUT_FEATURES = 2048
K = 32
NC = 2
NS = 16
NW = NC * NS
RPW = OUT_FEATURES // NW
L = 16
IW = IN_FEATURES // 2

TO = 256


def _compiler_params():
    cp = pltpu.CompilerParams()
    if "needs_layout_passes" in pltpu.CompilerParams.__dataclass_fields__:
        cp = dataclasses.replace(cp, needs_layout_passes=False)
    return cp


def _sc_scatter_weights_packed(idx, w):
    mesh = plsc.VectorSubcoreMesh(core_axis_name="c", subcore_axis_name="s")

    @functools.partial(
        pl.kernel,
        out_type=jax.ShapeDtypeStruct((OUT_FEATURES, IW), jnp.int32),
        mesh=mesh,
        compiler_params=_compiler_params(),
        scratch_types=[
            pltpu.VMEM((RPW, K), jnp.int32),
            pltpu.VMEM((RPW, K), jnp.float32),
            pltpu.VMEM((IN_FEATURES,), jnp.float32),
            pltpu.VMEM((IW,), jnp.int32),
            pltpu.VMEM((IW,), jnp.int32),
            pltpu.SemaphoreType.DMA,
            pltpu.SemaphoreType.DMA,
        ],
    )
    def body(idx_hbm, w_hbm, s_hbm, idx_v, w_v, fb_v, pb0_v, pb1_v,
             sem0, sem1):
        pbufs = (pb0_v, pb1_v)
        sems = (sem0, sem1)
        wid = lax.axis_index("s") * NC + lax.axis_index("c")
        base = wid * RPW
        pltpu.sync_copy(idx_hbm.at[pl.ds(base, RPW)], idx_v)
        pltpu.sync_copy(w_hbm.at[pl.ds(base, RPW)], w_v)

        zf = jnp.zeros((L,), jnp.float32)
        zi = jnp.zeros((L,), jnp.int32)

        @pl.loop(0, IN_FEATURES, step=2 * L)
        def _(c):
            fb_v[pl.ds(c, L)] = zf
            fb_v[pl.ds(c + L, L)] = zf
            pb0_v[pl.ds(lax.div(c, 2), L)] = zi
            pb1_v[pl.ds(lax.div(c, 2), L)] = zi

        def halves(r):
            return [idx_v[r, pl.ds(h * L, L)] for h in range(K // L)]

        def build_row(r, p):
            pb = pbufs[p]
            ivs = halves(r)
            wvs = [w_v[r, pl.ds(h * L, L)] for h in range(K // L)]
            for iv, wv in zip(ivs, wvs):
                plsc.addupdate_scatter(fb_v, [iv], wv)
            for iv in ivs:
                vals = plsc.load_gather(fb_v, [iv])
                plsc.store_scatter(fb_v, [iv], zf)
                u = plsc.bitcast(vals, jnp.int32)
                lsb = (u >> 16) & 1
                bits = ((u + 0x7FFF + lsb) >> 16) & 0xFFFF
                word = iv & (IW - 1)
                hi = iv >> 11
                shifted = jnp.where(hi == 1, bits << 16, bits)
                for parity in range(2):
                    m = hi == parity
                    cur = plsc.load_gather(pb, [word])
                    plsc.store_scatter(pb, [word], cur | shifted, mask=m)

        def unbuild_row(r, p):
            for iv in halves(r):
                plsc.store_scatter(pbufs[p], [iv & (IW - 1)], zi)

        def start_dma(r, p):
            pltpu.async_copy(pbufs[p], s_hbm.at[base + r], sems[p])

        def wait_dma(r, p):
            pltpu.make_async_copy(pbufs[p], s_hbm.at[base + r],
                                  sems[p]).wait()

        build_row(0, 0)
        start_dma(0, 0)
        build_row(1, 1)
        start_dma(1, 1)

        @pl.loop(2, RPW, step=2)
        def _(r0):
            for p in range(2):
                r = r0 + p
                wait_dma(r - 2, p)
                unbuild_row(r - 2, p)
                build_row(r, p)
                start_dma(r, p)

        wait_dma(RPW - 2, 0)
        wait_dma(RPW - 1, 1)

    return body(idx, w)


def _tc_matmul(x, s_packed, bias_row):
    NT = (((1,), (1,)), ((), ()))

    def body(x_ref, s_ref, b_ref, o_ref, xbf_ref):
        @pl.when(pl.program_id(0) == 0)
        def _():
            xbf_ref[...] = x_ref[...].astype(jnp.bfloat16)

        u = s_ref[...]
        lo = lax.bitcast_convert_type(u << 16, jnp.float32)
        hi = lax.bitcast_convert_type(u & jnp.int32(-65536), jnp.float32)
        sb = jnp.concatenate([lo.astype(jnp.bfloat16),
                              hi.astype(jnp.bfloat16)], axis=1)
        acc = lax.dot_general(xbf_ref[...], sb, NT,
                              preferred_element_type=jnp.float32)
        o_ref[...] = acc + b_ref[...]

    return pl.pallas_call(
        body,
        grid=(OUT_FEATURES // TO,),
        in_specs=[
            pl.BlockSpec((B, IN_FEATURES), lambda j: (0, 0)),
            pl.BlockSpec((TO, IW), lambda j: (j, 0)),
            pl.BlockSpec((1, TO), lambda j: (0, j)),
        ],
        out_specs=pl.BlockSpec((B, TO), lambda j: (0, j)),
        out_shape=jax.ShapeDtypeStruct((B, OUT_FEATURES), jnp.float32),
        scratch_shapes=[pltpu.VMEM((B, IN_FEATURES), jnp.bfloat16)],
    )(x, s_packed, bias_row)


def kernel(x, indices, weight, bias):
    idx = indices.astype(jnp.int32)
    s_packed = _sc_scatter_weights_packed(idx, weight.astype(jnp.float32))
    return _tc_matmul(x, s_packed,
                      bias.astype(jnp.float32).reshape(1, OUT_FEATURES))


# speedup vs baseline: 1.3594x; 1.0319x over previous
import dataclasses
import functools

import jax
import jax.numpy as jnp
from jax import lax
from jax.experimental import pallas as pl
from jax.experimental.pallas import tpu as pltpu
from jax.experimental.pallas import tpu_sc as plsc

B = 1024
IN_FEATURES = 4096
OUT_FEATURES = 2048
K = 32
NC = 2
NS = 16
NW = NC * NS
RPW = OUT_FEATURES // NW
L = 16
IW = IN_FEATURES // 2

TO = 256


def _compiler_params():
    cp = pltpu.CompilerParams()
    if "needs_layout_passes" in pltpu.CompilerParams.__dataclass_fields__:
        cp = dataclasses.replace(cp, needs_layout_passes=False)
    return cp


def _sc_scatter_weights_packed(idx_and_w):
    mesh = plsc.VectorSubcoreMesh(core_axis_name="c", subcore_axis_name="s")

    @functools.partial(
        pl.kernel,
        out_type=jax.ShapeDtypeStruct((OUT_FEATURES, IW), jnp.int32),
        mesh=mesh,
        compiler_params=_compiler_params(),
        scratch_types=[
            pltpu.VMEM((RPW, 2 * K), jnp.int32),
            pltpu.VMEM((IN_FEATURES,), jnp.float32),
            pltpu.VMEM((IW,), jnp.int32),
            pltpu.VMEM((IW,), jnp.int32),
            pltpu.SemaphoreType.DMA,
            pltpu.SemaphoreType.DMA,
        ],
    )
    def body(iw_hbm, s_hbm, iw_v, fb_v, pb0_v, pb1_v, sem0, sem1):
        pbufs = (pb0_v, pb1_v)
        sems = (sem0, sem1)
        wid = lax.axis_index("s") * NC + lax.axis_index("c")
        base = wid * RPW
        pltpu.sync_copy(iw_hbm.at[pl.ds(base, RPW)], iw_v)

        zf = jnp.zeros((L,), jnp.float32)
        zi = jnp.zeros((L,), jnp.int32)

        @pl.loop(0, IN_FEATURES, step=2 * L)
        def _(c):
            fb_v[pl.ds(c, L)] = zf
            fb_v[pl.ds(c + L, L)] = zf
            pb0_v[pl.ds(lax.div(c, 2), L)] = zi
            pb1_v[pl.ds(lax.div(c, 2), L)] = zi

        def halves(r):
            return [iw_v[r, pl.ds(h * L, L)] for h in range(K // L)]

        def build_row(r, p):
            pb = pbufs[p]
            ivs = halves(r)
            wvs = [plsc.bitcast(iw_v[r, pl.ds(K + h * L, L)], jnp.float32)
                   for h in range(K // L)]
            for iv, wv in zip(ivs, wvs):
                plsc.addupdate_scatter(fb_v, [iv], wv)
            for iv in ivs:
                vals = plsc.load_gather(fb_v, [iv])
                plsc.store_scatter(fb_v, [iv], zf)
                u = plsc.bitcast(vals, jnp.int32)
                lsb = (u >> 16) & 1
                bits = ((u + 0x7FFF + lsb) >> 16) & 0xFFFF
                word = iv & (IW - 1)
                hi = iv >> 11
                shifted = jnp.where(hi == 1, bits << 16, bits)
                for parity in range(2):
                    m = hi == parity
                    cur = plsc.load_gather(pb, [word])
                    plsc.store_scatter(pb, [word], cur | shifted, mask=m)

        def unbuild_row(r, p):
            for iv in halves(r):
                plsc.store_scatter(pbufs[p], [iv & (IW - 1)], zi)

        def start_dma(r, p):
            pltpu.async_copy(pbufs[p], s_hbm.at[base + r], sems[p])

        def wait_dma(r, p):
            pltpu.make_async_copy(pbufs[p], s_hbm.at[base + r],
                                  sems[p]).wait()

        build_row(0, 0)
        start_dma(0, 0)
        build_row(1, 1)
        start_dma(1, 1)

        @pl.loop(2, RPW, step=2)
        def _(r0):
            for p in range(2):
                r = r0 + p
                wait_dma(r - 2, p)
                unbuild_row(r - 2, p)
                build_row(r, p)
                start_dma(r, p)

        wait_dma(RPW - 2, 0)
        wait_dma(RPW - 1, 1)

    return body(idx_and_w)


def _tc_matmul(x, s_packed, bias_row):
    NT = (((1,), (1,)), ((), ()))

    def body(x_ref, s_ref, b_ref, o_ref, xbf_ref):
        @pl.when(pl.program_id(0) == 0)
        def _():
            xbf_ref[...] = x_ref[...].astype(jnp.bfloat16)

        u = s_ref[...]
        lo = lax.bitcast_convert_type(u << 16, jnp.float32)
        hi = lax.bitcast_convert_type(u & jnp.int32(-65536), jnp.float32)
        sb = jnp.concatenate([lo.astype(jnp.bfloat16),
                              hi.astype(jnp.bfloat16)], axis=1)
        acc = lax.dot_general(xbf_ref[...], sb, NT,
                              preferred_element_type=jnp.float32)
        o_ref[...] = acc + b_ref[...]

    return pl.pallas_call(
        body,
        grid=(OUT_FEATURES // TO,),
        in_specs=[
            pl.BlockSpec((B, IN_FEATURES), lambda j: (0, 0)),
            pl.BlockSpec((TO, IW), lambda j: (j, 0)),
            pl.BlockSpec((1, TO), lambda j: (0, j)),
        ],
        out_specs=pl.BlockSpec((B, TO), lambda j: (0, j)),
        out_shape=jax.ShapeDtypeStruct((B, OUT_FEATURES), jnp.float32),
        scratch_shapes=[pltpu.VMEM((B, IN_FEATURES), jnp.bfloat16)],
    )(x, s_packed, bias_row)


def kernel(x, indices, weight, bias):
    idx = indices.astype(jnp.int32)
    w_bits = lax.bitcast_convert_type(weight.astype(jnp.float32), jnp.int32)
    idx_and_w = jnp.concatenate([idx, w_bits], axis=1)
    s_packed = _sc_scatter_weights_packed(idx_and_w)
    return _tc_matmul(x, s_packed,
                      bias.astype(jnp.float32).reshape(1, OUT_FEATURES))
